# Initial kernel scaffold; baseline (speedup 1.0000x reference)
import jax
import jax.numpy as jnp
from jax.experimental import pallas as pl


def kernel(xi, edge_src, edge_dst, species):
    raise NotImplementedError("write your pallas kernel here")


# speedup vs baseline: 1.8628x; 1.8628x over previous
import functools

import jax
import jax.numpy as jnp
from jax import lax
from jax.experimental import pallas as pl
from jax.experimental.pallas import tpu as pltpu
from jax.experimental.pallas import tpu_sc as plsc

N_NODES = 10000
N_EDGES = 320000
D_FEAT = 128

_NC = 2
_NS = 16
_NW = _NC * _NS

_B2 = 2 * N_EDGES
_PER_W = _B2 // _NW
_CHUNK = 400
_NCHUNK = _PER_W // _CHUNK


def _make_gather():
    mesh = plsc.VectorSubcoreMesh(core_axis_name="c", subcore_axis_name="s")

    @functools.partial(
        pl.kernel,
        mesh=mesh,
        out_type=jax.ShapeDtypeStruct((_B2, D_FEAT), jnp.float32),
        scratch_types=[
            pltpu.VMEM((_CHUNK,), jnp.int32),
            pltpu.VMEM((_CHUNK, D_FEAT), jnp.float32),
            pltpu.SemaphoreType.DMA,
        ],
    )
    def gather_kernel(xi_hbm, idx_hbm, out_hbm, idx_v, rows_v, sem):
        wid = lax.axis_index("s") * _NC + lax.axis_index("c")
        base = wid * _PER_W

        def chunk_body(j, carry):
            off = base + j * _CHUNK
            pltpu.sync_copy(idx_hbm.at[pl.ds(off, _CHUNK)], idx_v)
            pltpu.async_copy(xi_hbm.at[idx_v], rows_v, sem).wait()
            pltpu.sync_copy(rows_v, out_hbm.at[pl.ds(off, _CHUNK)])
            return carry

        lax.fori_loop(0, _NCHUNK, chunk_body, 0)

    return gather_kernel


_gather = _make_gather()


def kernel(xi, edge_src, edge_dst, species):
    del species
    idx = jnp.stack(
        [edge_src.astype(jnp.int32), edge_dst.astype(jnp.int32)], axis=1
    ).reshape(_B2)
    out_flat = _gather(xi, idx)
    return out_flat.reshape(N_EDGES, 2 * D_FEAT)


# speedup vs baseline: 1.9586x; 1.0514x over previous
import functools

import jax
import jax.numpy as jnp
from jax import lax
from jax.experimental import pallas as pl
from jax.experimental.pallas import tpu as pltpu
from jax.experimental.pallas import tpu_sc as plsc

N_NODES = 10000
N_EDGES = 320000
D_FEAT = 128

_NC = 2
_NS = 16
_NW = _NC * _NS

_B2 = 2 * N_EDGES
_PER_W = _B2 // _NW
_CHUNK = 400
_NCHUNK = _PER_W // _CHUNK


def _make_gather():
    mesh = plsc.VectorSubcoreMesh(core_axis_name="c", subcore_axis_name="s")

    @functools.partial(
        pl.kernel,
        mesh=mesh,
        out_type=jax.ShapeDtypeStruct((_B2, D_FEAT), jnp.float32),
        scratch_types=[
            pltpu.VMEM((_CHUNK,), jnp.int32),
            pltpu.VMEM((_CHUNK, D_FEAT), jnp.float32),
            pltpu.VMEM((_CHUNK, D_FEAT), jnp.float32),
            pltpu.SemaphoreType.DMA,
            pltpu.SemaphoreType.DMA,
            pltpu.SemaphoreType.DMA,
        ],
    )
    def gather_kernel(xi_hbm, idx_hbm, out_hbm, idx_v, rows0, rows1,
                      sem_g, sem_s0, sem_s1):
        wid = lax.axis_index("s") * _NC + lax.axis_index("c")
        base = wid * _PER_W
        rows = (rows0, rows1)
        sems = (sem_s0, sem_s1)

        def pair_body(i, carry):
            for b in range(2):
                j = i * 2 + b
                off = base + j * _CHUNK

                @pl.when(i >= 1)
                def _drain_store(b=b):
                    pltpu.make_async_copy(
                        rows[b], out_hbm.at[pl.ds(0, _CHUNK)], sems[b]
                    ).wait()

                pltpu.sync_copy(idx_hbm.at[pl.ds(off, _CHUNK)], idx_v)
                pltpu.async_copy(xi_hbm.at[idx_v], rows[b], sem_g).wait()
                pltpu.async_copy(rows[b], out_hbm.at[pl.ds(off, _CHUNK)], sems[b])
            return carry

        lax.fori_loop(0, _NCHUNK // 2, pair_body, 0)
        for b in range(2):
            pltpu.make_async_copy(
                rows[b], out_hbm.at[pl.ds(0, _CHUNK)], sems[b]
            ).wait()

    return gather_kernel


_gather = _make_gather()


def kernel(xi, edge_src, edge_dst, species):
    del species
    idx = jnp.stack(
        [edge_src.astype(jnp.int32), edge_dst.astype(jnp.int32)], axis=1
    ).reshape(_B2)
    out_flat = _gather(xi, idx)
    return out_flat.reshape(N_EDGES, 2 * D_FEAT)


# speedup vs baseline: 1.9881x; 1.0151x over previous
import functools

import jax
import jax.numpy as jnp
from jax import lax
from jax.experimental import pallas as pl
from jax.experimental.pallas import tpu as pltpu
from jax.experimental.pallas import tpu_sc as plsc

N_NODES = 10000
N_EDGES = 320000
D_FEAT = 128

_NC = 2
_NS = 16
_NW = _NC * _NS

_B2 = 2 * N_EDGES
_PER_W = _B2 // _NW
_CHUNK = 200
_NCHUNK = _PER_W // _CHUNK
_NBUF = 4


def _make_gather():
    mesh = plsc.VectorSubcoreMesh(core_axis_name="c", subcore_axis_name="s")

    @functools.partial(
        pl.kernel,
        mesh=mesh,
        out_type=jax.ShapeDtypeStruct((_B2, D_FEAT), jnp.float32),
        scratch_types=[
            pltpu.VMEM((_PER_W,), jnp.int32),
        ]
        + [pltpu.VMEM((_CHUNK, D_FEAT), jnp.float32)] * _NBUF
        + [pltpu.SemaphoreType.DMA] * (2 * _NBUF),
    )
    def gather_kernel(xi_hbm, idx_hbm, out_hbm, idx_all, *bufs):
        rows = bufs[:_NBUF]
        sem_g = bufs[_NBUF:2 * _NBUF]
        sem_s = bufs[2 * _NBUF:]

        wid = lax.axis_index("s") * _NC + lax.axis_index("c")
        base = wid * _PER_W

        pltpu.sync_copy(idx_hbm.at[pl.ds(base, _PER_W)], idx_all)

        def gather_start(j, b):
            pltpu.async_copy(
                xi_hbm.at[idx_all.at[pl.ds(j * _CHUNK, _CHUNK)]],
                rows[b], sem_g[b],
            )

        for jj in range(2):
            gather_start(jj, jj)

        def quad_body(i, carry):
            for b in range(_NBUF):
                j = i * _NBUF + b
                bn = (b + 2) % _NBUF
                off = base + j * _CHUNK

                @pl.when(j >= 2)
                def _drain_store(bn=bn):
                    pltpu.make_async_copy(
                        rows[bn], out_hbm.at[pl.ds(0, _CHUNK)], sem_s[bn]
                    ).wait()

                @pl.when(j + 2 < _NCHUNK)
                def _next_gather(j=j, bn=bn):
                    gather_start(j + 2, bn)

                pltpu.make_async_copy(
                    xi_hbm.at[idx_all.at[pl.ds(0, _CHUNK)]], rows[b], sem_g[b]
                ).wait()
                pltpu.async_copy(rows[b], out_hbm.at[pl.ds(off, _CHUNK)], sem_s[b])
            return carry

        lax.fori_loop(0, _NCHUNK // _NBUF, quad_body, 0)
        for j in (_NCHUNK - 2, _NCHUNK - 1):
            pltpu.make_async_copy(
                rows[j % _NBUF], out_hbm.at[pl.ds(0, _CHUNK)], sem_s[j % _NBUF]
            ).wait()

    return gather_kernel


_gather = _make_gather()


def kernel(xi, edge_src, edge_dst, species):
    del species
    idx = jnp.stack(
        [edge_src.astype(jnp.int32), edge_dst.astype(jnp.int32)], axis=1
    ).reshape(_B2)
    out_flat = _gather(xi, idx)
    return out_flat.reshape(N_EDGES, 2 * D_FEAT)
